# baseline (speedup 1.0000x reference)
import functools

import jax
import jax.numpy as jnp
from jax import lax
from jax.experimental import pallas as pl
from jax.experimental.pallas import tpu as pltpu
from jax.experimental.pallas import tpu_sc as plsc

_B = 16
_LW = 4096
_LB = 2048
_LANES = 16
_HALF = _LW // 2
_CH = 512
_NCH = _HALF // _CH


def _sc_weight_sums_body(weight_hbm, wlens_hbm, out_hbm,
                         buf0, buf1, len_v, acc_v, sem0, sem1):
    nc = plsc.get_sparse_core_info().num_cores
    wid = lax.axis_index("s") * nc + lax.axis_index("c")
    r = wid % _B
    h = wid // _B
    half_base = h * _HALF

    bufs = (buf0, buf1)
    sems = (sem0, sem1)

    copies = [None, None]
    copies[0] = pltpu.async_copy(
        weight_hbm.at[r, pl.ds(half_base, _CH)], buf0, sem0)
    pltpu.sync_copy(wlens_hbm.at[r], len_v)
    len_vec = len_v[...]

    lane = lax.broadcasted_iota(jnp.int32, (_LANES,), 0)
    zeros = jnp.zeros((_LANES,), jnp.float32)
    accs = (zeros, zeros, zeros, zeros)

    for g in range(_NCH):
        nxt = (g + 1) % 2
        if g + 1 < _NCH:
            copies[nxt] = pltpu.async_copy(
                weight_hbm.at[r, pl.ds(half_base + (g + 1) * _CH, _CH)],
                bufs[nxt], sems[nxt])
        copies[g % 2].wait()
        buf = bufs[g % 2]
        chunk_base = half_base + g * _CH

        def body(j, a, buf=buf, chunk_base=chunk_base):
            base = j * (4 * _LANES)
            out = []
            for k in range(4):
                v = buf[pl.ds(base + k * _LANES, _LANES)]
                pos = chunk_base + base + k * _LANES + lane
                out.append(a[k] + jnp.where(pos < len_vec, v, 0.0))
            return tuple(out)

        accs = lax.fori_loop(0, _CH // (4 * _LANES), body, accs)

    acc_v[...] = (accs[0] + accs[1]) + (accs[2] + accs[3])
    pltpu.sync_copy(acc_v, out_hbm.at[wid])


_sc_weight_sums = functools.partial(
    pl.kernel,
    out_type=jax.ShapeDtypeStruct((2 * _B, _LANES), jnp.float32),
    mesh=plsc.VectorSubcoreMesh(core_axis_name="c", subcore_axis_name="s"),
    scratch_types=[
        pltpu.VMEM((_CH,), jnp.float32),
        pltpu.VMEM((_CH,), jnp.float32),
        pltpu.VMEM((_LANES,), jnp.int32),
        pltpu.VMEM((_LANES,), jnp.float32),
        pltpu.SemaphoreType.DMA,
        pltpu.SemaphoreType.DMA,
    ],
)(_sc_weight_sums_body)


def _tc_encode_kernel(partials_ref, bias_ref, wlen_ref, blen_ref,
                      W_w_ref, b_w_ref, W_b_ref, b_b_ref,
                      W_enc_ref, b_enc_ref, out_ref):
    psums = jnp.sum(partials_ref[...], axis=1, keepdims=True)
    s_w = psums[:_B, :] + psums[_B:, :]

    blen = blen_ref[...]
    mask_b = jax.lax.broadcasted_iota(jnp.int32, (_B, _LB), 1) < blen
    s_b = jnp.sum(jnp.where(mask_b, bias_ref[...], 0.0), axis=1, keepdims=True)

    lwf = wlen_ref[...].astype(jnp.float32)
    lbf = blen.astype(jnp.float32)

    emb_w = jax.nn.sigmoid(s_w * W_w_ref[...][None, :] + lwf * b_w_ref[...][None, :])
    emb_b = jax.nn.sigmoid(s_b * W_b_ref[...][None, :] + lbf * b_b_ref[...][None, :])

    emb = jnp.concatenate([emb_w, emb_b], axis=1)
    enc = jnp.dot(emb, W_enc_ref[...], preferred_element_type=jnp.float32)
    out_ref[...] = jnp.maximum(enc + b_enc_ref[...][None, :], 0.0)


def kernel(weight, bias, weight_parameters, bias_parameters, W_w, b_w, W_b, b_b, W_enc, b_enc):
    B = weight.shape[0]
    H = W_enc.shape[1]
    wlen = weight_parameters.astype(jnp.int32)
    blen = bias_parameters.astype(jnp.int32)
    wlen_b = jnp.broadcast_to(wlen[:, None], (_B, _LANES))

    partials = _sc_weight_sums(weight, wlen_b)

    return pl.pallas_call(
        _tc_encode_kernel,
        out_shape=jax.ShapeDtypeStruct((B, H), jnp.float32),
    )(partials, bias, wlen.reshape(B, 1), blen.reshape(B, 1),
      W_w, b_w, W_b, b_b, W_enc, b_enc)


# speedup vs baseline: 1.0229x; 1.0229x over previous
import functools

import jax
import jax.numpy as jnp
from jax import lax
from jax.experimental import pallas as pl
from jax.experimental.pallas import tpu as pltpu
from jax.experimental.pallas import tpu_sc as plsc

_B = 16
_LW = 4096
_LB = 2048
_LANES = 16
_HALF = _LW // 2


def _sc_weight_sums_body(weight_hbm, wlens_hbm, out_hbm, row_v, len_v, acc_v):
    nc = plsc.get_sparse_core_info().num_cores
    wid = lax.axis_index("s") * nc + lax.axis_index("c")
    r = wid % _B
    h = wid // _B

    pltpu.sync_copy(weight_hbm.at[r, pl.ds(h * _HALF, _HALF)], row_v)
    pltpu.sync_copy(wlens_hbm.at[r], len_v)
    len_vec = len_v[...]

    lane = lax.broadcasted_iota(jnp.int32, (_LANES,), 0)
    half_base = h * _HALF

    def body(j, accs):
        base = j * (4 * _LANES)
        out = []
        for k in range(4):
            v = row_v[pl.ds(base + k * _LANES, _LANES)]
            pos = half_base + base + k * _LANES + lane
            out.append(accs[k] + jnp.where(pos < len_vec, v, 0.0))
        return tuple(out)

    zeros = jnp.zeros((_LANES,), jnp.float32)
    accs = lax.fori_loop(0, _HALF // (4 * _LANES), body,
                         (zeros, zeros, zeros, zeros))
    acc_v[...] = (accs[0] + accs[1]) + (accs[2] + accs[3])
    pltpu.sync_copy(acc_v, out_hbm.at[wid])


_sc_weight_sums = functools.partial(
    pl.kernel,
    out_type=jax.ShapeDtypeStruct((2 * _B, _LANES), jnp.float32),
    mesh=plsc.VectorSubcoreMesh(core_axis_name="c", subcore_axis_name="s"),
    scratch_types=[
        pltpu.VMEM((_HALF,), jnp.float32),
        pltpu.VMEM((_LANES,), jnp.int32),
        pltpu.VMEM((_LANES,), jnp.float32),
    ],
)(_sc_weight_sums_body)


def _tc_encode_kernel(partials_ref, bias_ref, wlen_ref, blen_ref,
                      W_w_ref, b_w_ref, W_b_ref, b_b_ref,
                      W_enc_ref, b_enc_ref, out_ref):
    psums = jnp.sum(partials_ref[...], axis=1, keepdims=True)
    s_w = psums[:_B, :] + psums[_B:, :]

    blen = blen_ref[...]
    mask_b = jax.lax.broadcasted_iota(jnp.int32, (_B, _LB), 1) < blen
    s_b = jnp.sum(jnp.where(mask_b, bias_ref[...], 0.0), axis=1, keepdims=True)

    lwf = wlen_ref[...].astype(jnp.float32)
    lbf = blen.astype(jnp.float32)

    emb_w = jax.nn.sigmoid(s_w * W_w_ref[...][None, :] + lwf * b_w_ref[...][None, :])
    emb_b = jax.nn.sigmoid(s_b * W_b_ref[...][None, :] + lbf * b_b_ref[...][None, :])

    emb = jnp.concatenate([emb_w, emb_b], axis=1)
    enc = jnp.dot(emb, W_enc_ref[...], preferred_element_type=jnp.float32)
    out_ref[...] = jnp.maximum(enc + b_enc_ref[...][None, :], 0.0)


def kernel(weight, bias, weight_parameters, bias_parameters, W_w, b_w, W_b, b_b, W_enc, b_enc):
    B = weight.shape[0]
    H = W_enc.shape[1]
    wlen = weight_parameters.astype(jnp.int32)
    blen = bias_parameters.astype(jnp.int32)
    wlen_b = jnp.broadcast_to(wlen[:, None], (_B, _LANES))

    partials = _sc_weight_sums(weight, wlen_b)

    return pl.pallas_call(
        _tc_encode_kernel,
        out_shape=jax.ShapeDtypeStruct((B, H), jnp.float32),
    )(partials, bias, wlen.reshape(B, 1), blen.reshape(B, 1),
      W_w, b_w, W_b, b_b, W_enc, b_enc)
